# baseline (speedup 1.0000x reference)
import functools

import jax
import jax.numpy as jnp
from jax import lax
from jax.experimental import pallas as pl
from jax.experimental.pallas import tpu as pltpu
from jax.experimental.pallas import tpu_sc as plsc

_EPS = 1e-12
_NCH = 12
_ROW_BLK = 8
_SB = 8


def _sigma_body(u_ref, w_ref, sig_ref, acc_ref, ssq_ref):
    i = pl.program_id(0)

    @pl.when(i == 0)
    def _init():
        acc_ref[...] = jnp.zeros_like(acc_ref)
        ssq_ref[0] = 0.0

    w = w_ref[...]
    t1 = lax.dot_general(w, u_ref[...],
                         dimension_numbers=(((1,), (1,)), ((), ())),
                         preferred_element_type=jnp.float32)
    ssq_ref[0] += jnp.sum(t1 * t1)
    acc_ref[...] += t1 * w

    @pl.when(i == pl.num_programs(0) - 1)
    def _fin():
        s = jnp.sum(acc_ref[...], axis=0, keepdims=True)
        s_sq = jnp.sum(s * s)
        n1 = jnp.sqrt(ssq_ref[0])
        d1 = n1 + _EPS
        t2_sq = s_sq / (d1 * d1)
        n2 = jnp.sqrt(t2_sq)
        sig_ref[0, 0] = t2_sq / (n2 + _EPS)


def _sigma_call(table, u):
    n_cls, f = table.shape
    grid = n_cls // _ROW_BLK
    return pl.pallas_call(
        _sigma_body,
        grid=(grid,),
        in_specs=[
            pl.BlockSpec((1, f), lambda i: (0, 0)),
            pl.BlockSpec((_ROW_BLK, f), lambda i: (i, 0)),
        ],
        out_specs=pl.BlockSpec(memory_space=pltpu.SMEM),
        out_shape=jax.ShapeDtypeStruct((1, 1), jnp.float32),
        scratch_shapes=[
            pltpu.VMEM((_ROW_BLK, f), jnp.float32),
            pltpu.SMEM((1,), jnp.float32),
        ],
        compiler_params=pltpu.CompilerParams(
            dimension_semantics=("arbitrary",),
        ),
    )(u, table)


@functools.lru_cache(maxsize=None)
def _sc_gather_fn(n_tbl_rows, ch, n_units):
    try:
        info = plsc.get_sparse_core_info()
        nc, ns = info.num_cores, info.num_subcores
    except Exception:
        nc, ns = 2, 16
    nw = nc * ns
    upw = n_units // nw
    nsb = upw // _SB

    mesh = plsc.VectorSubcoreMesh(
        core_axis_name="c", subcore_axis_name="s",
        num_cores=nc, num_subcores=ns,
    )

    @functools.partial(
        pl.kernel,
        mesh=mesh,
        out_type=jax.ShapeDtypeStruct((n_units, ch), jnp.float32),
        scratch_types=[
            pltpu.VMEM((upw,), jnp.int32),
            pltpu.VMEM((_SB, ch), jnp.float32),
            pltpu.VMEM((_SB, ch), jnp.float32),
            pltpu.SemaphoreType.DMA,
            pltpu.SemaphoreType.DMA,
        ],
    )
    def gather_k(tbl_hbm, idx_hbm, out_hbm, idx_v, b0, b1, s0, s1):
        wid = lax.axis_index("s") * nc + lax.axis_index("c")
        base = wid * upw
        pltpu.sync_copy(idx_hbm.at[pl.ds(base, upw)], idx_v)
        bufs = (b0, b1)
        sems = (s0, s1)
        copies = [
            pltpu.make_async_copy(
                tbl_hbm.at[idx_v.at[pl.ds(sb * _SB, _SB)]],
                bufs[sb % 2],
                sems[sb % 2],
            )
            for sb in range(nsb)
        ]
        copies[0].start()
        for sb in range(nsb):
            if sb + 1 < nsb:
                copies[sb + 1].start()
            copies[sb].wait()
            pltpu.sync_copy(bufs[sb % 2],
                            out_hbm.at[pl.ds(base + sb * _SB, _SB)])

    return gather_k


def _cond_body(sig_ref, t_ref, e_ref, o_ref):
    o_ref[...] = t_ref[...] + e_ref[...] * (1.0 / sig_ref[0, 0])


def _cond_call(sig, tensor_flat, emb):
    b, f = tensor_flat.shape
    grid = b // _ROW_BLK
    return pl.pallas_call(
        _cond_body,
        grid=(grid,),
        in_specs=[
            pl.BlockSpec(memory_space=pltpu.SMEM),
            pl.BlockSpec((_ROW_BLK, f), lambda i: (i, 0)),
            pl.BlockSpec((_ROW_BLK, f), lambda i: (i, 0)),
        ],
        out_specs=pl.BlockSpec((_ROW_BLK, f), lambda i: (i, 0)),
        out_shape=jax.ShapeDtypeStruct((b, f), jnp.float32),
        compiler_params=pltpu.CompilerParams(
            dimension_semantics=("parallel",),
        ),
    )(sig, tensor_flat, emb)


def kernel(tensor, labels, table, u):
    b, h, w_, c = tensor.shape
    n_cls, f = table.shape
    ch = f // _NCH
    n_units = b * _NCH

    sig = _sigma_call(table, u)

    labels32 = labels.astype(jnp.int32)
    idx = (labels32[:, None] * _NCH
           + jnp.arange(_NCH, dtype=jnp.int32)).reshape(n_units)
    tbl_rs = table.reshape(n_cls * _NCH, ch)
    emb_rs = _sc_gather_fn(n_cls * _NCH, ch, n_units)(tbl_rs, idx)

    out = _cond_call(sig, tensor.reshape(b, f), emb_rs.reshape(b, f))
    return out.reshape(b, h, w_, c)


# speedup vs baseline: 1.9711x; 1.9711x over previous
import functools

import jax
import jax.numpy as jnp
from jax import lax
from jax.experimental import pallas as pl
from jax.experimental.pallas import tpu as pltpu
from jax.experimental.pallas import tpu_sc as plsc

_EPS = 1e-12
_NCH = 12
_ROW_BLK = 8
_CW = 128
_NACC = 8
_CND_BLK = 8
_SB = 8


def _sigma_body(u_ref, w_ref, sig_ref, acc_ref, ssq_ref):
    i = pl.program_id(0)

    @pl.when(i == 0)
    def _init():
        acc_ref[...] = jnp.zeros_like(acc_ref)
        ssq_ref[0] = 0.0

    f = w_ref.shape[1]
    nchk = f // _CW
    accs = [None] * _NACC
    for k in range(nchk):
        sl = pl.ds(k * _CW, _CW)
        c = w_ref[:, sl] * u_ref[:, sl]
        j = k % _NACC
        accs[j] = c if accs[j] is None else accs[j] + c
    while len(accs) > 1:
        accs = [a + b for a, b in zip(accs[::2], accs[1::2])]
    t1 = jnp.sum(accs[0], axis=1, keepdims=True)
    ssq_ref[0] += jnp.sum(t1 * t1)
    t1b = jnp.broadcast_to(t1, (t1.shape[0], _CW))
    for k in range(nchk):
        sl = pl.ds(k * _CW, _CW)
        acc_ref[:, sl] += t1b * w_ref[:, sl]

    @pl.when(i == pl.num_programs(0) - 1)
    def _fin():
        s = jnp.sum(acc_ref[...], axis=0, keepdims=True)
        s_sq = jnp.sum(s * s)
        n1 = jnp.sqrt(ssq_ref[0])
        d1 = n1 + _EPS
        t2_sq = s_sq / (d1 * d1)
        n2 = jnp.sqrt(t2_sq)
        sig_ref[0, 0] = t2_sq / (n2 + _EPS)


def _sigma_call(table, u):
    n_cls, f = table.shape
    grid = n_cls // _ROW_BLK
    u8 = jnp.broadcast_to(u, (_ROW_BLK, f))
    return pl.pallas_call(
        _sigma_body,
        grid=(grid,),
        in_specs=[
            pl.BlockSpec((_ROW_BLK, f), lambda i: (0, 0)),
            pl.BlockSpec((_ROW_BLK, f), lambda i: (i, 0)),
        ],
        out_specs=pl.BlockSpec(memory_space=pltpu.SMEM),
        out_shape=jax.ShapeDtypeStruct((1, 1), jnp.float32),
        scratch_shapes=[
            pltpu.VMEM((_ROW_BLK, f), jnp.float32),
            pltpu.SMEM((1,), jnp.float32),
        ],
        compiler_params=pltpu.CompilerParams(
            dimension_semantics=("arbitrary",),
        ),
    )(u8, table)


@functools.lru_cache(maxsize=None)
def _sc_gather_fn(n_cls, f, b):
    try:
        info = plsc.get_sparse_core_info()
        nc, ns = info.num_cores, info.num_subcores
    except Exception:
        nc, ns = 2, 16
    nw = nc * ns
    ch = f // _NCH
    ngrp = b // _SB
    nbat = ngrp * _NCH
    bpw = nbat // nw

    mesh = plsc.VectorSubcoreMesh(
        core_axis_name="c", subcore_axis_name="s",
        num_cores=nc, num_subcores=ns,
    )

    @functools.partial(
        pl.kernel,
        mesh=mesh,
        out_type=jax.ShapeDtypeStruct((b, f), jnp.float32),
        scratch_types=[
            pltpu.VMEM((_SB,), jnp.int32),
            pltpu.VMEM((_SB, ch), jnp.float32),
            pltpu.VMEM((_SB, ch), jnp.float32),
            pltpu.SemaphoreType.DMA,
            pltpu.SemaphoreType.DMA,
        ],
    )
    def gather_k(tbl_hbm, lbl_hbm, out_hbm, idx_v, b0, b1, s0, s1):
        wid = lax.axis_index("s") * nc + lax.axis_index("c")
        grp = wid % ngrp
        cb = (wid // ngrp) * bpw
        pltpu.sync_copy(lbl_hbm.at[pl.ds(grp * _SB, _SB)], idx_v)
        bufs = (b0, b1)
        sems = (s0, s1)
        copies = [
            pltpu.make_async_copy(
                tbl_hbm.at[idx_v, pl.ds((cb + k) * ch, ch)],
                bufs[k % 2],
                sems[k % 2],
            )
            for k in range(bpw)
        ]
        copies[0].start()
        for k in range(bpw):
            if k + 1 < bpw:
                copies[k + 1].start()
            copies[k].wait()
            pltpu.sync_copy(
                bufs[k % 2],
                out_hbm.at[pl.ds(grp * _SB, _SB),
                           pl.ds((cb + k) * ch, ch)])

    return gather_k


def _cond_body(sig_ref, t_ref, e_ref, o_ref):
    o_ref[...] = t_ref[...] + e_ref[...] * (1.0 / sig_ref[0, 0])


def _cond_call(sig, tensor_flat, emb):
    b, f = tensor_flat.shape
    grid = b // _CND_BLK
    return pl.pallas_call(
        _cond_body,
        grid=(grid,),
        in_specs=[
            pl.BlockSpec(memory_space=pltpu.SMEM),
            pl.BlockSpec((_CND_BLK, f), lambda i: (i, 0)),
            pl.BlockSpec((_CND_BLK, f), lambda i: (i, 0)),
        ],
        out_specs=pl.BlockSpec((_CND_BLK, f), lambda i: (i, 0)),
        out_shape=jax.ShapeDtypeStruct((b, f), jnp.float32),
        compiler_params=pltpu.CompilerParams(
            dimension_semantics=("parallel",),
        ),
    )(sig, tensor_flat, emb)


def kernel(tensor, labels, table, u):
    b, h, w_, c = tensor.shape
    n_cls, f = table.shape

    sig = _sigma_call(table, u)

    labels32 = labels.astype(jnp.int32)
    emb = _sc_gather_fn(n_cls, f, b)(table, labels32)

    out = _cond_call(sig, tensor.reshape(b, f), emb)
    return out.reshape(b, h, w_, c)


# speedup vs baseline: 2.3401x; 1.1872x over previous
import functools

import jax
import jax.numpy as jnp
from jax import lax
from jax.experimental import pallas as pl
from jax.experimental.pallas import tpu as pltpu
from jax.experimental.pallas import tpu_sc as plsc

_EPS = 1e-12
_NCH = 12
_ROW_BLK = 40
_CW = 128
_NACC = 8
_CND_BLK = 8
_SB = 8


def _sigma_body(u_ref, w_ref, sig_ref, acc_ref, ssq_ref):
    i = pl.program_id(0)

    @pl.when(i == 0)
    def _init():
        acc_ref[...] = jnp.zeros_like(acc_ref)
        ssq_ref[0] = 0.0

    f = w_ref.shape[1]
    nchk = f // _CW
    accs = [None] * _NACC
    for k in range(nchk):
        sl = pl.ds(k * _CW, _CW)
        c = w_ref[:, sl] * u_ref[:, sl]
        j = k % _NACC
        accs[j] = c if accs[j] is None else accs[j] + c
    while len(accs) > 1:
        accs = [a + b for a, b in zip(accs[::2], accs[1::2])]
    t1 = jnp.sum(accs[0], axis=1, keepdims=True)
    ssq_ref[0] += jnp.sum(t1 * t1)
    t1b = jnp.broadcast_to(t1, (t1.shape[0], _CW))
    for k in range(nchk):
        sl = pl.ds(k * _CW, _CW)
        acc_ref[:, sl] += t1b * w_ref[:, sl]

    @pl.when(i == pl.num_programs(0) - 1)
    def _fin():
        s = jnp.sum(acc_ref[...], axis=0, keepdims=True)
        s_sq = jnp.sum(s * s)
        n1 = jnp.sqrt(ssq_ref[0])
        d1 = n1 + _EPS
        t2_sq = s_sq / (d1 * d1)
        n2 = jnp.sqrt(t2_sq)
        sig_ref[0, 0] = t2_sq / (n2 + _EPS)


def _sigma_call(table, u):
    n_cls, f = table.shape
    grid = n_cls // _ROW_BLK
    u8 = jnp.broadcast_to(u, (_ROW_BLK, f))
    return pl.pallas_call(
        _sigma_body,
        grid=(grid,),
        in_specs=[
            pl.BlockSpec((_ROW_BLK, f), lambda i: (0, 0)),
            pl.BlockSpec((_ROW_BLK, f), lambda i: (i, 0)),
        ],
        out_specs=pl.BlockSpec(memory_space=pltpu.SMEM),
        out_shape=jax.ShapeDtypeStruct((1, 1), jnp.float32),
        scratch_shapes=[
            pltpu.VMEM((_ROW_BLK, f), jnp.float32),
            pltpu.SMEM((1,), jnp.float32),
        ],
        compiler_params=pltpu.CompilerParams(
            dimension_semantics=("arbitrary",),
        ),
    )(u8, table)


@functools.lru_cache(maxsize=None)
def _sc_gather_fn(n_cls, f, b):
    try:
        info = plsc.get_sparse_core_info()
        nc, ns = info.num_cores, info.num_subcores
    except Exception:
        nc, ns = 2, 16
    nw = nc * ns
    ch = f // _NCH
    ngrp = b // _SB
    nbat = ngrp * _NCH
    bpw = nbat // nw

    mesh = plsc.VectorSubcoreMesh(
        core_axis_name="c", subcore_axis_name="s",
        num_cores=nc, num_subcores=ns,
    )

    @functools.partial(
        pl.kernel,
        mesh=mesh,
        out_type=jax.ShapeDtypeStruct((b, f), jnp.float32),
        scratch_types=[
            pltpu.VMEM((_SB,), jnp.int32),
            pltpu.VMEM((_SB, ch), jnp.float32),
            pltpu.VMEM((_SB, ch), jnp.float32),
            pltpu.SemaphoreType.DMA,
            pltpu.SemaphoreType.DMA,
        ],
    )
    def gather_k(tbl_hbm, lbl_hbm, out_hbm, idx_v, b0, b1, s0, s1):
        wid = lax.axis_index("s") * nc + lax.axis_index("c")
        grp = wid % ngrp
        cb = (wid // ngrp) * bpw
        pltpu.sync_copy(lbl_hbm.at[pl.ds(grp * _SB, _SB)], idx_v)
        bufs = (b0, b1)
        sems = (s0, s1)
        copies = [
            pltpu.make_async_copy(
                tbl_hbm.at[idx_v, pl.ds((cb + k) * ch, ch)],
                bufs[k % 2],
                sems[k % 2],
            )
            for k in range(bpw)
        ]
        copies[0].start()
        for k in range(bpw):
            if k + 1 < bpw:
                copies[k + 1].start()
            copies[k].wait()
            pltpu.sync_copy(
                bufs[k % 2],
                out_hbm.at[pl.ds(grp * _SB, _SB),
                           pl.ds((cb + k) * ch, ch)])

    return gather_k


def _cond_body(sig_ref, t_ref, e_ref, o_ref):
    o_ref[...] = t_ref[...] + e_ref[...] * (1.0 / sig_ref[0, 0])


def _cond_call(sig, tensor_flat, emb):
    b, f = tensor_flat.shape
    grid = b // _CND_BLK
    return pl.pallas_call(
        _cond_body,
        grid=(grid,),
        in_specs=[
            pl.BlockSpec(memory_space=pltpu.SMEM),
            pl.BlockSpec((_CND_BLK, f), lambda i: (i, 0)),
            pl.BlockSpec((_CND_BLK, f), lambda i: (i, 0)),
        ],
        out_specs=pl.BlockSpec((_CND_BLK, f), lambda i: (i, 0)),
        out_shape=jax.ShapeDtypeStruct((b, f), jnp.float32),
        compiler_params=pltpu.CompilerParams(
            dimension_semantics=("parallel",),
        ),
    )(sig, tensor_flat, emb)


def kernel(tensor, labels, table, u):
    b, h, w_, c = tensor.shape
    n_cls, f = table.shape

    sig = _sigma_call(table, u)

    labels32 = labels.astype(jnp.int32)
    emb = _sc_gather_fn(n_cls, f, b)(table, labels32)

    out = _cond_call(sig, tensor.reshape(b, f), emb)
    return out.reshape(b, h, w_, c)


# speedup vs baseline: 2.7050x; 1.1559x over previous
import functools

import jax
import jax.numpy as jnp
from jax import lax
from jax.experimental import pallas as pl
from jax.experimental.pallas import tpu as pltpu
from jax.experimental.pallas import tpu_sc as plsc

_EPS = 1e-12
_NCH = 12
_ROW_BLK = 40
_CW = 128
_NACC = 8
_CND_BLK = 8
_SB = 8


def _sigma_body(u_ref, w_ref, sig_ref, acc_ref, ssq_ref):
    i = pl.program_id(0)

    @pl.when(i == 0)
    def _init():
        acc_ref[...] = jnp.zeros_like(acc_ref)
        ssq_ref[0] = 0.0

    f = w_ref.shape[1]
    nchk = f // _CW
    accs = [None] * _NACC
    for k in range(nchk):
        sl = pl.ds(k * _CW, _CW)
        c = w_ref[:, sl] * u_ref[:, sl]
        j = k % _NACC
        accs[j] = c if accs[j] is None else accs[j] + c
    while len(accs) > 1:
        accs = [a + b for a, b in zip(accs[::2], accs[1::2])]
    t1 = jnp.sum(accs[0], axis=1, keepdims=True)
    ssq_ref[0] += jnp.sum(t1 * t1)
    t1b = jnp.broadcast_to(t1, (t1.shape[0], _CW))
    for k in range(nchk):
        sl = pl.ds(k * _CW, _CW)
        acc_ref[:, sl] += t1b * w_ref[:, sl]

    @pl.when(i == pl.num_programs(0) - 1)
    def _fin():
        s = jnp.sum(acc_ref[...], axis=0, keepdims=True)
        s_sq = jnp.sum(s * s)
        n1 = jnp.sqrt(ssq_ref[0])
        d1 = n1 + _EPS
        t2_sq = s_sq / (d1 * d1)
        n2 = jnp.sqrt(t2_sq)
        sig_ref[0, 0] = t2_sq / (n2 + _EPS)


def _sigma_call(table, u):
    n_cls, f = table.shape
    grid = n_cls // _ROW_BLK
    u8 = jnp.broadcast_to(u, (_ROW_BLK, f))
    return pl.pallas_call(
        _sigma_body,
        grid=(grid,),
        in_specs=[
            pl.BlockSpec((_ROW_BLK, f), lambda i: (0, 0)),
            pl.BlockSpec((_ROW_BLK, f), lambda i: (i, 0)),
        ],
        out_specs=pl.BlockSpec(memory_space=pltpu.SMEM),
        out_shape=jax.ShapeDtypeStruct((1, 1), jnp.float32),
        scratch_shapes=[
            pltpu.VMEM((_ROW_BLK, f), jnp.float32),
            pltpu.SMEM((1,), jnp.float32),
        ],
        compiler_params=pltpu.CompilerParams(
            dimension_semantics=("arbitrary",),
        ),
    )(u8, table)


@functools.lru_cache(maxsize=None)
def _sc_gather_fn(n_cls, f, b):
    try:
        info = plsc.get_sparse_core_info()
        nc, ns = info.num_cores, info.num_subcores
    except Exception:
        nc, ns = 2, 16
    nw = nc * ns
    ch = f // _NCH
    ngrp = b // _SB
    nbat = ngrp * _NCH
    bpw = nbat // nw

    mesh = plsc.VectorSubcoreMesh(
        core_axis_name="c", subcore_axis_name="s",
        num_cores=nc, num_subcores=ns,
    )

    @functools.partial(
        pl.kernel,
        mesh=mesh,
        out_type=jax.ShapeDtypeStruct((b, f), jnp.float32),
        scratch_types=[
            pltpu.VMEM((_SB,), jnp.int32),
            pltpu.VMEM((_SB, ch), jnp.float32),
            pltpu.VMEM((_SB, ch), jnp.float32),
            pltpu.SemaphoreType.DMA,
            pltpu.SemaphoreType.DMA,
        ],
    )
    def gather_k(tbl_hbm, lbl_hbm, out_hbm, idx_v, b0, b1, s0, s1):
        wid = lax.axis_index("s") * nc + lax.axis_index("c")
        grp = wid % ngrp
        cb = (wid // ngrp) * bpw
        pltpu.sync_copy(lbl_hbm.at[pl.ds(grp * _SB, _SB)], idx_v)
        bufs = (b0, b1)
        sems = (s0, s1)
        copies = [
            pltpu.make_async_copy(
                tbl_hbm.at[idx_v, pl.ds((cb + k) * ch, ch)],
                bufs[k % 2],
                sems[k % 2],
            )
            for k in range(bpw)
        ]
        copies[0].start()
        for k in range(bpw):
            if k + 1 < bpw:
                copies[k + 1].start()
            copies[k].wait()
            pltpu.sync_copy(
                bufs[k % 2],
                out_hbm.at[pl.ds(grp * _SB, _SB),
                           pl.ds((cb + k) * ch, ch)])

    return gather_k


def _cond_body(sig_ref, t_ref, e_ref, o_ref):
    inv = 1.0 / sig_ref[0, 0]
    e4 = e_ref[...].reshape(o_ref.shape)
    o_ref[...] = t_ref[...] + e4 * inv


def _cond_call(sig, tensor, emb):
    b, h, w_, c = tensor.shape
    f = h * w_ * c
    grid = b // _CND_BLK
    return pl.pallas_call(
        _cond_body,
        grid=(grid,),
        in_specs=[
            pl.BlockSpec(memory_space=pltpu.SMEM),
            pl.BlockSpec((_CND_BLK, h, w_, c), lambda i: (i, 0, 0, 0)),
            pl.BlockSpec((_CND_BLK, f), lambda i: (i, 0)),
        ],
        out_specs=pl.BlockSpec((_CND_BLK, h, w_, c), lambda i: (i, 0, 0, 0)),
        out_shape=jax.ShapeDtypeStruct((b, h, w_, c), jnp.float32),
        compiler_params=pltpu.CompilerParams(
            dimension_semantics=("parallel",),
        ),
    )(sig, tensor, emb)


def kernel(tensor, labels, table, u):
    b, h, w_, c = tensor.shape
    n_cls, f = table.shape

    sig = _sigma_call(table, u)

    labels32 = labels.astype(jnp.int32)
    emb = _sc_gather_fn(n_cls, f, b)(table, labels32)

    return _cond_call(sig, tensor, emb)


# speedup vs baseline: 3.4383x; 1.2711x over previous
import functools

import jax
import jax.numpy as jnp
from jax import lax
from jax.experimental import pallas as pl
from jax.experimental.pallas import tpu as pltpu
from jax.experimental.pallas import tpu_sc as plsc

_EPS = 1e-12
_NCH = 12
_ROW_BLK = 40
_CW = 128
_NACC = 8
_CND_BLK = 8
_SB = 8


def _sigma_body(u_ref, w_ref, sig_ref, acc_ref, ssq_ref):
    i = pl.program_id(0)

    @pl.when(i == 0)
    def _init():
        acc_ref[...] = jnp.zeros_like(acc_ref)
        ssq_ref[0] = 0.0

    f = w_ref.shape[1]
    nchk = f // _CW
    accs = [None] * _NACC
    for k in range(nchk):
        sl = pl.ds(k * _CW, _CW)
        c = w_ref[:, sl] * u_ref[:, sl]
        j = k % _NACC
        accs[j] = c if accs[j] is None else accs[j] + c
    while len(accs) > 1:
        accs = [a + b for a, b in zip(accs[::2], accs[1::2])]
    t1 = jnp.sum(accs[0], axis=1, keepdims=True)
    ssq_ref[0] += jnp.sum(t1 * t1)
    t1b = jnp.broadcast_to(t1, (t1.shape[0], _CW))
    for k in range(nchk):
        sl = pl.ds(k * _CW, _CW)
        acc_ref[:, sl] += t1b * w_ref[:, sl]

    @pl.when(i == pl.num_programs(0) - 1)
    def _fin():
        s = jnp.sum(acc_ref[...], axis=0, keepdims=True)
        s_sq = jnp.sum(s * s)
        n1 = jnp.sqrt(ssq_ref[0])
        d1 = n1 + _EPS
        t2_sq = s_sq / (d1 * d1)
        n2 = jnp.sqrt(t2_sq)
        sig_ref[0, 0] = t2_sq / (n2 + _EPS)


def _sigma_call(table, u):
    n_cls, f = table.shape
    grid = n_cls // _ROW_BLK
    u8 = jnp.broadcast_to(u, (_ROW_BLK, f))
    return pl.pallas_call(
        _sigma_body,
        grid=(grid,),
        in_specs=[
            pl.BlockSpec((_ROW_BLK, f), lambda i: (0, 0)),
            pl.BlockSpec((_ROW_BLK, f), lambda i: (i, 0)),
        ],
        out_specs=pl.BlockSpec(memory_space=pltpu.SMEM),
        out_shape=jax.ShapeDtypeStruct((1, 1), jnp.float32),
        scratch_shapes=[
            pltpu.VMEM((_ROW_BLK, f), jnp.float32),
            pltpu.SMEM((1,), jnp.float32),
        ],
        compiler_params=pltpu.CompilerParams(
            dimension_semantics=("arbitrary",),
        ),
    )(u8, table)


@functools.lru_cache(maxsize=None)
def _sc_gather_fn(n_cls, f, b):
    try:
        info = plsc.get_sparse_core_info()
        nc, ns = info.num_cores, info.num_subcores
    except Exception:
        nc, ns = 2, 16
    nw = nc * ns
    ch = f // _NCH
    ngrp = b // _SB
    nbat = ngrp * _NCH
    bpw = nbat // nw

    mesh = plsc.VectorSubcoreMesh(
        core_axis_name="c", subcore_axis_name="s",
        num_cores=nc, num_subcores=ns,
    )

    @functools.partial(
        pl.kernel,
        mesh=mesh,
        out_type=jax.ShapeDtypeStruct((b, f), jnp.float32),
        scratch_types=[
            pltpu.VMEM((_SB,), jnp.int32),
            pltpu.VMEM((_SB, ch), jnp.float32),
            pltpu.VMEM((_SB, ch), jnp.float32),
            pltpu.SemaphoreType.DMA,
            pltpu.SemaphoreType.DMA,
        ],
    )
    def gather_k(tbl_hbm, lbl_hbm, out_hbm, idx_v, b0, b1, s0, s1):
        wid = lax.axis_index("s") * nc + lax.axis_index("c")
        grp = wid % ngrp
        cb = (wid // ngrp) * bpw
        pltpu.sync_copy(lbl_hbm.at[pl.ds(grp * _SB, _SB)], idx_v)
        bufs = (b0, b1)
        sems = (s0, s1)
        copies = [
            pltpu.make_async_copy(
                tbl_hbm.at[idx_v, pl.ds((cb + k) * ch, ch)],
                bufs[k % 2],
                sems[k % 2],
            )
            for k in range(bpw)
        ]
        copies[0].start()
        for k in range(bpw):
            if k + 1 < bpw:
                copies[k + 1].start()
            copies[k].wait()
            pltpu.sync_copy(
                bufs[k % 2],
                out_hbm.at[pl.ds(grp * _SB, _SB),
                           pl.ds((cb + k) * ch, ch)])

    return gather_k


def _cond_body(sig_ref, t_ref, e_ref, o_ref):
    inv = 1.0 / sig_ref[0, 0]
    n_w = t_ref.shape[1]
    c = t_ref.shape[3]
    for w_i in range(n_w):
        o_ref[0, w_i] = (t_ref[0, w_i]
                         + e_ref[:, pl.ds(w_i * c, c)] * inv)


def _cond_call(sig, tensor_t, emb):
    h, w_, b, c = tensor_t.shape
    return pl.pallas_call(
        _cond_body,
        grid=(h,),
        in_specs=[
            pl.BlockSpec(memory_space=pltpu.SMEM),
            pl.BlockSpec((1, w_, b, c), lambda i: (i, 0, 0, 0)),
            pl.BlockSpec((b, w_ * c), lambda i: (0, i)),
        ],
        out_specs=pl.BlockSpec((1, w_, b, c), lambda i: (i, 0, 0, 0)),
        out_shape=jax.ShapeDtypeStruct((h, w_, b, c), jnp.float32),
        compiler_params=pltpu.CompilerParams(
            dimension_semantics=("parallel",),
        ),
    )(sig, tensor_t, emb)


def kernel(tensor, labels, table, u):
    b, h, w_, c = tensor.shape
    n_cls, f = table.shape

    sig = _sigma_call(table, u)

    labels32 = labels.astype(jnp.int32)
    emb = _sc_gather_fn(n_cls, f, b)(table, labels32)

    tensor_t = jnp.transpose(tensor, (1, 2, 0, 3))
    out_t = _cond_call(sig, tensor_t, emb)
    return jnp.transpose(out_t, (2, 0, 1, 3))


# speedup vs baseline: 3.6973x; 1.0753x over previous
import functools

import jax
import jax.numpy as jnp
from jax import lax
from jax.experimental import pallas as pl
from jax.experimental.pallas import tpu as pltpu
from jax.experimental.pallas import tpu_sc as plsc

_EPS = 1e-12
_NCH = 12
_ROW_BLK = 40
_CW = 128
_NACC = 8
_CND_BLK = 8
_SB = 8


def _sigma_body(u_ref, w_ref, sig_ref, acc_ref, ssq_ref):
    i = pl.program_id(0)

    @pl.when(i == 0)
    def _init():
        acc_ref[...] = jnp.zeros_like(acc_ref)
        ssq_ref[0] = 0.0

    f = w_ref.shape[1]
    nchk = f // _CW
    accs = [None] * _NACC
    for k in range(nchk):
        sl = pl.ds(k * _CW, _CW)
        c = w_ref[:, sl] * u_ref[:, sl]
        j = k % _NACC
        accs[j] = c if accs[j] is None else accs[j] + c
    while len(accs) > 1:
        accs = [a + b for a, b in zip(accs[::2], accs[1::2])]
    t1 = jnp.sum(accs[0], axis=1, keepdims=True)
    ssq_ref[0] += jnp.sum(t1 * t1)
    t1b = jnp.broadcast_to(t1, (t1.shape[0], _CW))
    for k in range(nchk):
        sl = pl.ds(k * _CW, _CW)
        acc_ref[:, sl] += t1b * w_ref[:, sl]

    @pl.when(i == pl.num_programs(0) - 1)
    def _fin():
        s = jnp.sum(acc_ref[...], axis=0, keepdims=True)
        s_sq = jnp.sum(s * s)
        n1 = jnp.sqrt(ssq_ref[0])
        d1 = n1 + _EPS
        t2_sq = s_sq / (d1 * d1)
        n2 = jnp.sqrt(t2_sq)
        sig_ref[0, 0] = t2_sq / (n2 + _EPS)


def _sigma_call(table, u):
    n_cls, f = table.shape
    grid = n_cls // _ROW_BLK
    return pl.pallas_call(
        _sigma_body,
        grid=(grid,),
        in_specs=[
            pl.BlockSpec((1, f), lambda i: (0, 0)),
            pl.BlockSpec((_ROW_BLK, f), lambda i: (i, 0)),
        ],
        out_specs=pl.BlockSpec(memory_space=pltpu.SMEM),
        out_shape=jax.ShapeDtypeStruct((1, 1), jnp.float32),
        scratch_shapes=[
            pltpu.VMEM((_ROW_BLK, f), jnp.float32),
            pltpu.SMEM((1,), jnp.float32),
        ],
        compiler_params=pltpu.CompilerParams(
            dimension_semantics=("arbitrary",),
        ),
    )(u, table)


@functools.lru_cache(maxsize=None)
def _sc_gather_fn(n_cls, f, b):
    try:
        info = plsc.get_sparse_core_info()
        nc, ns = info.num_cores, info.num_subcores
    except Exception:
        nc, ns = 2, 16
    nw = nc * ns
    ch = f // _NCH
    ngrp = b // _SB
    nbat = ngrp * _NCH
    bpw = nbat // nw

    mesh = plsc.VectorSubcoreMesh(
        core_axis_name="c", subcore_axis_name="s",
        num_cores=nc, num_subcores=ns,
    )

    @functools.partial(
        pl.kernel,
        mesh=mesh,
        out_type=jax.ShapeDtypeStruct((b, f), jnp.float32),
        scratch_types=[
            pltpu.VMEM((_SB,), jnp.int32),
            pltpu.VMEM((_SB, ch), jnp.float32),
            pltpu.VMEM((_SB, ch), jnp.float32),
            pltpu.SemaphoreType.DMA,
            pltpu.SemaphoreType.DMA,
        ],
    )
    def gather_k(tbl_hbm, lbl_hbm, out_hbm, idx_v, b0, b1, s0, s1):
        wid = lax.axis_index("s") * nc + lax.axis_index("c")
        grp = wid % ngrp
        cb = (wid // ngrp) * bpw
        pltpu.sync_copy(lbl_hbm.at[pl.ds(grp * _SB, _SB)], idx_v)
        bufs = (b0, b1)
        sems = (s0, s1)
        copies = [
            pltpu.make_async_copy(
                tbl_hbm.at[idx_v, pl.ds((cb + k) * ch, ch)],
                bufs[k % 2],
                sems[k % 2],
            )
            for k in range(bpw)
        ]
        copies[0].start()
        for k in range(bpw):
            if k + 1 < bpw:
                copies[k + 1].start()
            copies[k].wait()
            pltpu.sync_copy(
                bufs[k % 2],
                out_hbm.at[pl.ds(grp * _SB, _SB),
                           pl.ds((cb + k) * ch, ch)])

    return gather_k


def _cond_body(sig_ref, t_ref, e_ref, o_ref):
    inv = 1.0 / sig_ref[0, 0]
    n_w = t_ref.shape[1]
    c = t_ref.shape[3]
    for w_i in range(n_w):
        o_ref[0, w_i] = (t_ref[0, w_i]
                         + e_ref[:, pl.ds(w_i * c, c)] * inv)


def _cond_call(sig, tensor_t, emb):
    h, w_, b, c = tensor_t.shape
    return pl.pallas_call(
        _cond_body,
        grid=(h,),
        in_specs=[
            pl.BlockSpec(memory_space=pltpu.SMEM),
            pl.BlockSpec((1, w_, b, c), lambda i: (i, 0, 0, 0)),
            pl.BlockSpec((b, w_ * c), lambda i: (0, i)),
        ],
        out_specs=pl.BlockSpec((1, w_, b, c), lambda i: (i, 0, 0, 0)),
        out_shape=jax.ShapeDtypeStruct((h, w_, b, c), jnp.float32),
        compiler_params=pltpu.CompilerParams(
            dimension_semantics=("parallel",),
        ),
    )(sig, tensor_t, emb)


def kernel(tensor, labels, table, u):
    b, h, w_, c = tensor.shape
    n_cls, f = table.shape

    sig = _sigma_call(table, u)

    labels32 = labels.astype(jnp.int32)
    emb = _sc_gather_fn(n_cls, f, b)(table, labels32)

    tensor_t = jnp.transpose(tensor, (1, 2, 0, 3))
    out_t = _cond_call(sig, tensor_t, emb)
    return jnp.transpose(out_t, (2, 0, 1, 3))
